# baseline (speedup 1.0000x reference)
import jax
import jax.numpy as jnp
from jax.experimental import pallas as pl
from jax.experimental.pallas import tpu as pltpu

N = 10000
TM = 1024
TMR = 400
TMP = 2000
TM1 = 400


def _pass1_body(a_ref, x_ref, w1_ref, w_ref, o_ref, abf_ref, xw1_ref):
    @pl.when(pl.program_id(0) == 0)
    def _():
        xw1_ref[...] = jnp.dot(
            x_ref[...], w1_ref[...],
            preferred_element_type=jnp.float32).astype(jnp.bfloat16)

    a = a_ref[...].astype(jnp.bfloat16)
    abf_ref[...] = a
    h = jnp.maximum(
        jnp.dot(a, xw1_ref[...], preferred_element_type=jnp.float32), 0.0)
    o_ref[...] = jnp.dot(h.astype(jnp.bfloat16), w_ref[...],
                         preferred_element_type=jnp.float32).astype(jnp.bfloat16)


def _pass2_body(a_ref, b_ref, w_ref, o_ref):
    h = jnp.maximum(
        jnp.dot(a_ref[...], b_ref[...], preferred_element_type=jnp.float32), 0.0)
    o_ref[...] = jnp.dot(h.astype(jnp.bfloat16), w_ref[...],
                         preferred_element_type=jnp.float32).astype(jnp.bfloat16)


def _pass3_body(a_ref, b_ref, zm_ref, zs_ref):
    zz = jnp.dot(a_ref[...], b_ref[...], preferred_element_type=jnp.float32)
    zm_ref[...] = zz[:, :128]
    zs_ref[...] = jnp.clip(zz[:, 128:], -5.0, 2.0)


def _recon_dec_body(z_ref, zfull_ref, x_ref, dw_ref, db_ref, g1_ref, b1_ref,
                    fw_ref, fb_ref, g2_ref, b2_ref,
                    pw1_ref, pb1_ref, pw2_ref, pb2_ref,
                    sw1_ref, sb1_ref, sw2_ref, sb2_ref,
                    tw1_ref, tb1_ref, tw2_ref, tb2_ref,
                    o_ref, pos_ref, size_ref, tl_ref, tp_ref):
    o_ref[...] = jax.lax.dot_general(
        z_ref[...].astype(jnp.bfloat16), zfull_ref[...].astype(jnp.bfloat16),
        (((1,), (1,)), ((), ())), preferred_element_type=jnp.float32)
    dd = jnp.dot(x_ref[...], dw_ref[...],
                 preferred_element_type=jnp.float32) + db_ref[...]
    dm = jnp.mean(dd, axis=-1, keepdims=True)
    dv = jnp.mean((dd - dm) ** 2, axis=-1, keepdims=True)
    direct = jnp.maximum(
        (dd - dm) * jax.lax.rsqrt(dv + 1e-5) * g1_ref[...] + b1_ref[...], 0.0)
    fw = fw_ref[...]
    pre = (jnp.dot(z_ref[...], fw[:128, :], preferred_element_type=jnp.float32)
           + jnp.dot(direct, fw[128:, :], preferred_element_type=jnp.float32)
           + fb_ref[...])
    m = jnp.mean(pre, axis=-1, keepdims=True)
    v = jnp.mean((pre - m) ** 2, axis=-1, keepdims=True)
    fused = jnp.maximum(
        (pre - m) * jax.lax.rsqrt(v + 1e-5) * g2_ref[...] + b2_ref[...], 0.0)

    ph = jnp.maximum(jnp.dot(fused, pw1_ref[...],
                             preferred_element_type=jnp.float32) + pb1_ref[...], 0.0)
    pos_ref[...] = jnp.tanh(
        jnp.dot(ph, pw2_ref[...], preferred_element_type=jnp.float32) + pb2_ref[...])

    sh = jnp.maximum(jnp.dot(fused, sw1_ref[...],
                             preferred_element_type=jnp.float32) + sb1_ref[...], 0.0)
    size_ref[...] = jax.nn.sigmoid(
        jnp.dot(sh, sw2_ref[...], preferred_element_type=jnp.float32)
        + sb2_ref[...]) * 0.78 + 0.02

    th = jnp.maximum(jnp.dot(fused, tw1_ref[...],
                             preferred_element_type=jnp.float32) + tb1_ref[...], 0.0)
    tl = (jnp.dot(th, tw2_ref[...], preferred_element_type=jnp.float32)
          + tb2_ref[...])
    tl_ref[...] = tl
    e = jnp.exp(tl - jnp.max(tl, axis=-1, keepdims=True))
    tp_ref[...] = e / jnp.sum(e, axis=-1, keepdims=True)


def _const_spec(shape):
    return pl.BlockSpec(shape, lambda *_: tuple(0 for _ in shape))


def _rows_spec(tm, w):
    return pl.BlockSpec((tm, w), lambda i: (i, 0))


def _adj_pass(body, adj, b_op, extra_ws, out_widths, out_dtypes, tm=TM):
    n_out = len(out_widths)
    outs = pl.pallas_call(
        body,
        grid=(pl.cdiv(N, tm),),
        in_specs=[_rows_spec(tm, N), _const_spec(b_op.shape)]
        + [_const_spec(w.shape) for w in extra_ws],
        out_specs=[_rows_spec(tm, w) for w in out_widths],
        out_shape=[jax.ShapeDtypeStruct((N, w), dt)
                   for w, dt in zip(out_widths, out_dtypes)],
        compiler_params=pltpu.CompilerParams(
            dimension_semantics=("parallel",)),
    )(adj, b_op, *extra_ws)
    return outs if n_out > 1 else outs[0]


def kernel(x, adj, W1, W2, Wm, Ws, de_W, de_b, ln1_g, ln1_b, fu_W, fu_b,
           ln2_g, ln2_b, pW1, pb1, pW2, pb2, sW1, sb1, sW2, sb2,
           tW1, tb1, tW2, tb2):
    r = lambda v: v.reshape(1, -1)

    g1, adj_bf = pl.pallas_call(
        _pass1_body,
        grid=(N // TM1,),
        in_specs=[_rows_spec(TM1, N), _const_spec((N, 128)),
                  _const_spec((128, 256)), _const_spec((256, 128))],
        out_specs=[_rows_spec(TM1, 128), _rows_spec(TM1, N)],
        out_shape=[jax.ShapeDtypeStruct((N, 128), jnp.bfloat16),
                   jax.ShapeDtypeStruct((N, N), jnp.bfloat16)],
        scratch_shapes=[pltpu.VMEM((N, 256), jnp.bfloat16)],
        compiler_params=pltpu.CompilerParams(
            dimension_semantics=("arbitrary",)),
    )(adj, x, W1, W2.astype(jnp.bfloat16))
    wcat = jnp.concatenate([Wm, Ws], axis=1).astype(jnp.bfloat16)
    g2 = _adj_pass(_pass2_body, adj_bf, g1, [wcat], [256], [jnp.bfloat16])
    z_mean, z_log_std = _adj_pass(_pass3_body, adj_bf, g2, [], [128, 128],
                                  [jnp.float32, jnp.float32])

    adj_recon, pos, size, type_logits, type_pred = pl.pallas_call(
        _recon_dec_body,
        grid=(N // TMR,),
        in_specs=[_rows_spec(TMR, 128), _const_spec((N, 128)),
                  _rows_spec(TMR, 128),
                  _const_spec((128, 128)), _const_spec((1, 128)),
                  _const_spec((1, 128)), _const_spec((1, 128)),
                  _const_spec((256, 256)), _const_spec((1, 256)),
                  _const_spec((1, 256)), _const_spec((1, 256)),
                  _const_spec((256, 64)), _const_spec((1, 64)),
                  _const_spec((64, 2)), _const_spec((1, 2)),
                  _const_spec((256, 64)), _const_spec((1, 64)),
                  _const_spec((64, 2)), _const_spec((1, 2)),
                  _const_spec((256, 64)), _const_spec((1, 64)),
                  _const_spec((64, 14)), _const_spec((1, 14))],
        out_specs=[_rows_spec(TMR, N),
                   _rows_spec(TMR, 2), _rows_spec(TMR, 2),
                   _rows_spec(TMR, 14), _rows_spec(TMR, 14)],
        out_shape=[jax.ShapeDtypeStruct((N, N), jnp.float32),
                   jax.ShapeDtypeStruct((N, 2), jnp.float32),
                   jax.ShapeDtypeStruct((N, 2), jnp.float32),
                   jax.ShapeDtypeStruct((N, 14), jnp.float32),
                   jax.ShapeDtypeStruct((N, 14), jnp.float32)],
        compiler_params=pltpu.CompilerParams(
            dimension_semantics=("parallel",)),
    )(z_mean, z_mean, x, de_W, r(de_b), r(ln1_g), r(ln1_b),
      fu_W, r(fu_b), r(ln2_g), r(ln2_b),
      pW1, r(pb1), pW2, r(pb2), sW1, r(sb1), sW2, r(sb2),
      tW1, r(tb1), tW2, r(tb2))

    return (z_mean, z_mean, z_log_std, adj_recon, pos, size,
            type_logits, type_pred)
